# Initial kernel scaffold; baseline (speedup 1.0000x reference)
import jax
import jax.numpy as jnp
from jax.experimental import pallas as pl


def kernel(visual_tokens, scores):
    raise NotImplementedError("write your pallas kernel here")


# speedup vs baseline: 1.2606x; 1.2606x over previous
import functools

import jax
import jax.numpy as jnp
from jax import lax
from jax.experimental import pallas as pl
from jax.experimental.pallas import tpu as pltpu
from jax.experimental.pallas import tpu_sc as plsc

B, N, D = 8, 8192, 1024
K = N // 4
NC, NS, L = 2, 16, 16
BPC = B // NC
TPB = NS // BPC
RPT = K // TPB
GCH = 32
NV = N // L
MSB = -2147483648

_mesh = plsc.VectorSubcoreMesh(
    core_axis_name="c", subcore_axis_name="s", num_cores=NC, num_subcores=NS
)


def _keys16(sref, i):
    v = sref[pl.ds(i * L, L)]
    return v ^ (lax.shift_right_arithmetic(v, 31) & jnp.int32(0x7FFFFFFF))


_OUT_TYPE = (
    jax.ShapeDtypeStruct((B * K, D), jnp.float32),
    jax.ShapeDtypeStruct((B * K,), jnp.int32),
    jax.ShapeDtypeStruct((B * K,), jnp.int32),
)
_SCRATCH = [
    pltpu.VMEM((N,), jnp.int32),
    pltpu.VMEM((N,), jnp.int32),
    pltpu.VMEM((K + L,), jnp.int32),
    pltpu.VMEM((K + L,), jnp.int32),
    pltpu.VMEM((RPT,), jnp.int32),
    pltpu.VMEM((2, GCH, D), jnp.float32),
    pltpu.SemaphoreType.DMA,
    pltpu.SemaphoreType.DMA,
]


def _pruner_body(tok_hbm, scr_hbm, out_tok, out_idx, out_scr,
                 svmem, ubuf, oidx, oscr, idxg, rowbuf, sem_g, sem_w):
    c = lax.axis_index("c")
    s = lax.axis_index("s")

    @pl.when(s < BPC)
    def _select():
        b = c * BPC + s
        pltpu.sync_copy(scr_hbm.at[pl.ds(b * N, N)], svmem)

        def mkub(i, _):
            ubuf[pl.ds(i * L, L)] = _keys16(svmem, i) ^ MSB
            return 0
        lax.fori_loop(0, NV, mkub, 0)

        def step(j, carry):
            base, cnt, prefix = carry
            sh = 31 - j
            nv = (cnt + (L - 1)) // L
            lane = lax.iota(jnp.int32, L)

            def cbody(i, acc):
                v = ubuf[pl.ds(i * L, L)]
                valid = (i * L + lane) < cnt
                bit = lax.shift_right_logical(v, sh) & 1
                return acc + jnp.where(valid & (bit == 1), 1, 0)

            acc = lax.fori_loop(0, nv, cbody, jnp.zeros((L,), jnp.int32))
            c1 = jnp.sum(acc)
            want = jnp.where(base + c1 >= K, 1, 0)

            def kbody(i, o):
                v = ubuf[pl.ds(i * L, L)]
                valid = (i * L + lane) < cnt
                bit = lax.shift_right_logical(v, sh) & 1
                keep = valid & (bit == want)
                plsc.store_compressed(ubuf.at[pl.ds(o, L)], v, mask=keep)
                return o + jnp.sum(jnp.where(keep, 1, 0))

            newcnt = lax.fori_loop(0, nv, kbody, jnp.int32(0))
            base = jnp.where(want == 1, base, base + c1)
            return base, newcnt, prefix | (want << sh)

        base, _, t_ub = lax.fori_loop(
            0, 32, step, (jnp.int32(0), jnp.int32(N), jnp.int32(0))
        )
        t_key = t_ub ^ MSB
        need_eq = K - base

        lane = lax.iota(jnp.int32, L)

        def bbody(i, carry):
            o, e = carry
            key = _keys16(svmem, i)
            gt = key > t_key
            eq = key == t_key
            eqi = jnp.where(eq, 1, 0)
            excl = plsc.cumsum(eqi) - eqi
            keep = gt | (eq & (e + excl < need_eq))
            plsc.store_compressed(oidx.at[pl.ds(o, L)], i * L + lane, mask=keep)
            plsc.store_compressed(
                oscr.at[pl.ds(o, L)], svmem[pl.ds(i * L, L)], mask=keep
            )
            o = o + jnp.sum(jnp.where(keep, 1, 0))
            return o, e + jnp.sum(eqi)

        lax.fori_loop(0, NV, bbody, (jnp.int32(0), jnp.int32(0)))
        pltpu.sync_copy(oidx.at[pl.ds(0, K)], out_idx.at[pl.ds(b * K, K)])
        pltpu.sync_copy(oscr.at[pl.ds(0, K)], out_scr.at[pl.ds(b * K, K)])

    plsc.subcore_barrier()

    b2 = c * BPC + s // TPB
    row0 = b2 * K + (s % TPB) * RPT
    pltpu.sync_copy(out_idx.at[pl.ds(row0, RPT)], idxg)

    def gbody(i, _):
        idxg[pl.ds(i * L, L)] = idxg[pl.ds(i * L, L)] + b2 * N
        return 0
    lax.fori_loop(0, RPT // L, gbody, 0)

    def cbody(g, _):
        pltpu.async_copy(
            tok_hbm.at[idxg.at[pl.ds(g * GCH, GCH)]], rowbuf.at[0], sem_g
        ).wait()
        pltpu.sync_copy(rowbuf.at[0], out_tok.at[pl.ds(row0 + g * GCH, GCH)])
        return 0
    lax.fori_loop(0, RPT // GCH, cbody, 0)


_pruner = pl.kernel(
    _pruner_body,
    out_type=_OUT_TYPE,
    mesh=_mesh,
    scratch_types=_SCRATCH,
    compiler_params=pltpu.CompilerParams(needs_layout_passes=False),
)


def kernel(visual_tokens, scores):
    tok = visual_tokens.reshape(B * N, D)
    scr = lax.bitcast_convert_type(scores.reshape(B * N), jnp.int32)
    out_tok, out_idx, out_scr = _pruner(tok, scr)
    return (
        out_tok.reshape(B, K, D),
        out_idx.reshape(B, K),
        lax.bitcast_convert_type(out_scr, jnp.float32).reshape(B, K),
    )


# speedup vs baseline: 1.3543x; 1.0743x over previous
import functools

import jax
import jax.numpy as jnp
from jax import lax
from jax.experimental import pallas as pl
from jax.experimental.pallas import tpu as pltpu
from jax.experimental.pallas import tpu_sc as plsc

B, N, D = 8, 8192, 1024
K = N // 4
NC, NS, L = 2, 16, 16
BPC = B // NC
TPB = NS // BPC
RPT = K // TPB
GCH = 32
NV = N // L
MSB = -2147483648

_mesh = plsc.VectorSubcoreMesh(
    core_axis_name="c", subcore_axis_name="s", num_cores=NC, num_subcores=NS
)


def _keys16(sref, i):
    v = sref[pl.ds(i * L, L)]
    return v ^ (lax.shift_right_arithmetic(v, 31) & jnp.int32(0x7FFFFFFF))


_OUT_TYPE = (
    jax.ShapeDtypeStruct((B * K, D), jnp.float32),
    jax.ShapeDtypeStruct((B * K,), jnp.int32),
    jax.ShapeDtypeStruct((B * K,), jnp.int32),
)
_SCRATCH = [
    pltpu.VMEM((N,), jnp.int32),
    pltpu.VMEM((N,), jnp.int32),
    pltpu.VMEM((K + L,), jnp.int32),
    pltpu.VMEM((K + L,), jnp.int32),
    pltpu.VMEM((RPT,), jnp.int32),
    pltpu.VMEM((2, GCH, D), jnp.float32),
    pltpu.SemaphoreType.DMA,
    pltpu.SemaphoreType.DMA,
    pltpu.SemaphoreType.DMA,
]


def _pruner_body(tok_hbm, scr_hbm, out_tok, out_idx, out_scr,
                 svmem, ubuf, oidx, oscr, idxg, rowbuf, sem_g, sem_w0, sem_w1):
    c = lax.axis_index("c")
    s = lax.axis_index("s")

    @pl.when(s < BPC)
    def _select():
        b = c * BPC + s
        pltpu.sync_copy(scr_hbm.at[pl.ds(b * N, N)], svmem)

        def mkub(i, _):
            ubuf[pl.ds(i * L, L)] = _keys16(svmem, i) ^ MSB
            return 0
        lax.fori_loop(0, NV, mkub, 0)

        def step(j, carry):
            base, cnt, prefix = carry
            sh = 31 - j
            nv = (cnt + (L - 1)) // L
            lane = lax.iota(jnp.int32, L)

            def cbody(i, acc):
                v = ubuf[pl.ds(i * L, L)]
                valid = (i * L + lane) < cnt
                bit = lax.shift_right_logical(v, sh) & 1
                return acc + jnp.where(valid & (bit == 1), 1, 0)

            acc = lax.fori_loop(0, nv, cbody, jnp.zeros((L,), jnp.int32))
            c1 = jnp.sum(acc)
            want = jnp.where(base + c1 >= K, 1, 0)

            def kbody(i, o):
                v = ubuf[pl.ds(i * L, L)]
                valid = (i * L + lane) < cnt
                bit = lax.shift_right_logical(v, sh) & 1
                keep = valid & (bit == want)
                plsc.store_compressed(ubuf.at[pl.ds(o, L)], v, mask=keep)
                return o + jnp.sum(jnp.where(keep, 1, 0))

            newcnt = lax.fori_loop(0, nv, kbody, jnp.int32(0))
            base = jnp.where(want == 1, base, base + c1)
            return base, newcnt, prefix | (want << sh)

        base, _, t_ub = lax.fori_loop(
            0, 32, step, (jnp.int32(0), jnp.int32(N), jnp.int32(0))
        )
        t_key = t_ub ^ MSB
        need_eq = K - base

        lane = lax.iota(jnp.int32, L)

        def bbody(i, carry):
            o, e = carry
            key = _keys16(svmem, i)
            gt = key > t_key
            eq = key == t_key
            eqi = jnp.where(eq, 1, 0)
            excl = plsc.cumsum(eqi) - eqi
            keep = gt | (eq & (e + excl < need_eq))
            plsc.store_compressed(oidx.at[pl.ds(o, L)], i * L + lane, mask=keep)
            plsc.store_compressed(
                oscr.at[pl.ds(o, L)], svmem[pl.ds(i * L, L)], mask=keep
            )
            o = o + jnp.sum(jnp.where(keep, 1, 0))
            return o, e + jnp.sum(eqi)

        lax.fori_loop(0, NV, bbody, (jnp.int32(0), jnp.int32(0)))
        pltpu.sync_copy(oidx.at[pl.ds(0, K)], out_idx.at[pl.ds(b * K, K)])
        pltpu.sync_copy(oscr.at[pl.ds(0, K)], out_scr.at[pl.ds(b * K, K)])

    plsc.subcore_barrier()

    b2 = c * BPC + s // TPB
    row0 = b2 * K + (s % TPB) * RPT
    pltpu.sync_copy(out_idx.at[pl.ds(row0, RPT)], idxg)

    def gbody(i, _):
        idxg[pl.ds(i * L, L)] = idxg[pl.ds(i * L, L)] + b2 * N
        return 0
    lax.fori_loop(0, RPT // L, gbody, 0)

    sem_w = (sem_w0, sem_w1)

    def cbody(t, _):
        for par in range(2):
            g = t * 2 + par
            buf = rowbuf.at[par]

            @pl.when(g >= 2)
            def _drain():
                pltpu.make_async_copy(
                    buf, out_tok.at[pl.ds(row0, GCH)], sem_w[par]
                ).wait()

            pltpu.async_copy(
                tok_hbm.at[idxg.at[pl.ds(g * GCH, GCH)]], buf, sem_g
            ).wait()
            pltpu.async_copy(
                buf, out_tok.at[pl.ds(row0 + g * GCH, GCH)], sem_w[par]
            )
        return 0

    lax.fori_loop(0, RPT // GCH // 2, cbody, 0)
    for par in range(2):
        pltpu.make_async_copy(
            rowbuf.at[par], out_tok.at[pl.ds(row0, GCH)], sem_w[par]
        ).wait()


_pruner = pl.kernel(
    _pruner_body,
    out_type=_OUT_TYPE,
    mesh=_mesh,
    scratch_types=_SCRATCH,
    compiler_params=pltpu.CompilerParams(needs_layout_passes=False),
)


def kernel(visual_tokens, scores):
    tok = visual_tokens.reshape(B * N, D)
    scr = lax.bitcast_convert_type(scores.reshape(B * N), jnp.int32)
    out_tok, out_idx, out_scr = _pruner(tok, scr)
    return (
        out_tok.reshape(B, K, D),
        out_idx.reshape(B, K),
        lax.bitcast_convert_type(out_scr, jnp.float32).reshape(B, K),
    )


# speedup vs baseline: 1.3750x; 1.0153x over previous
import functools

import jax
import jax.numpy as jnp
from jax import lax
from jax.experimental import pallas as pl
from jax.experimental.pallas import tpu as pltpu
from jax.experimental.pallas import tpu_sc as plsc

B, N, D = 8, 8192, 1024
K = N // 4
NC, NS, L = 2, 16, 16
BPC = B // NC
TPB = NS // BPC
RPT = K // TPB
GCH = 16
NBUF = 4
NV = N // L
MSB = -2147483648

_mesh = plsc.VectorSubcoreMesh(
    core_axis_name="c", subcore_axis_name="s", num_cores=NC, num_subcores=NS
)


def _keys16(sref, i):
    v = sref[pl.ds(i * L, L)]
    return v ^ (lax.shift_right_arithmetic(v, 31) & jnp.int32(0x7FFFFFFF))


_OUT_TYPE = (
    jax.ShapeDtypeStruct((B * K, D), jnp.float32),
    jax.ShapeDtypeStruct((B * K,), jnp.int32),
    jax.ShapeDtypeStruct((B * K,), jnp.int32),
)
_SCRATCH = [
    pltpu.VMEM((N,), jnp.int32),
    pltpu.VMEM((N,), jnp.int32),
    pltpu.VMEM((K + L,), jnp.int32),
    pltpu.VMEM((K + L,), jnp.int32),
    pltpu.VMEM((RPT,), jnp.int32),
    pltpu.VMEM((NBUF, GCH, D), jnp.float32),
    pltpu.SemaphoreType.DMA((NBUF,)),
    pltpu.SemaphoreType.DMA((NBUF,)),
]


def _pruner_body(tok_hbm, scr_hbm, out_tok, out_idx, out_scr,
                 svmem, ubuf, oidx, oscr, idxg, rowbuf, sem_g, sem_w):
    c = lax.axis_index("c")
    s = lax.axis_index("s")

    @pl.when(s < BPC)
    def _select():
        b = c * BPC + s
        pltpu.sync_copy(scr_hbm.at[pl.ds(b * N, N)], svmem)

        def mkub(i, _):
            ubuf[pl.ds(i * L, L)] = _keys16(svmem, i) ^ MSB
            return 0
        lax.fori_loop(0, NV, mkub, 0)

        def step(j, carry):
            base, cnt, prefix = carry
            sh = 31 - j
            nv = (cnt + (L - 1)) // L
            lane = lax.iota(jnp.int32, L)

            def cbody(i, acc):
                v = ubuf[pl.ds(i * L, L)]
                valid = (i * L + lane) < cnt
                bit = lax.shift_right_logical(v, sh) & 1
                return acc + jnp.where(valid & (bit == 1), 1, 0)

            acc = lax.fori_loop(0, nv, cbody, jnp.zeros((L,), jnp.int32))
            c1 = jnp.sum(acc)
            want = jnp.where(base + c1 >= K, 1, 0)

            def kbody(i, o):
                v = ubuf[pl.ds(i * L, L)]
                valid = (i * L + lane) < cnt
                bit = lax.shift_right_logical(v, sh) & 1
                keep = valid & (bit == want)
                plsc.store_compressed(ubuf.at[pl.ds(o, L)], v, mask=keep)
                return o + jnp.sum(jnp.where(keep, 1, 0))

            newcnt = lax.fori_loop(0, nv, kbody, jnp.int32(0))
            base = jnp.where(want == 1, base, base + c1)
            return base, newcnt, prefix | (want << sh)

        base, _, t_ub = lax.fori_loop(
            0, 32, step, (jnp.int32(0), jnp.int32(N), jnp.int32(0))
        )
        t_key = t_ub ^ MSB
        need_eq = K - base

        lane = lax.iota(jnp.int32, L)

        def bbody(i, carry):
            o, e = carry
            key = _keys16(svmem, i)
            gt = key > t_key
            eq = key == t_key
            eqi = jnp.where(eq, 1, 0)
            excl = plsc.cumsum(eqi) - eqi
            keep = gt | (eq & (e + excl < need_eq))
            plsc.store_compressed(oidx.at[pl.ds(o, L)], i * L + lane, mask=keep)
            plsc.store_compressed(
                oscr.at[pl.ds(o, L)], svmem[pl.ds(i * L, L)], mask=keep
            )
            o = o + jnp.sum(jnp.where(keep, 1, 0))
            return o, e + jnp.sum(eqi)

        lax.fori_loop(0, NV, bbody, (jnp.int32(0), jnp.int32(0)))
        pltpu.sync_copy(oidx.at[pl.ds(0, K)], out_idx.at[pl.ds(b * K, K)])
        pltpu.sync_copy(oscr.at[pl.ds(0, K)], out_scr.at[pl.ds(b * K, K)])

    plsc.subcore_barrier()

    b2 = c * BPC + s // TPB
    row0 = b2 * K + (s % TPB) * RPT
    pltpu.sync_copy(out_idx.at[pl.ds(row0, RPT)], idxg)

    def gbody(i, _):
        idxg[pl.ds(i * L, L)] = idxg[pl.ds(i * L, L)] + b2 * N
        return 0
    lax.fori_loop(0, RPT // L, gbody, 0)

    NCH = RPT // GCH
    LOOK = 2

    def _gather(g, par):
        pltpu.async_copy(
            tok_hbm.at[idxg.at[pl.ds(g * GCH, GCH)]],
            rowbuf.at[par], sem_g.at[par],
        )

    def _wait_write(par):
        pltpu.make_async_copy(
            rowbuf.at[par], out_tok.at[pl.ds(row0, GCH)], sem_w.at[par]
        ).wait()

    for p in range(LOOK):
        _gather(p, p)

    def cbody(tt, _):
        for j in range(NBUF):
            t = tt * NBUF + j

            @pl.when((t >= LOOK) & (t < NCH))
            def _reuse():
                _wait_write((j + LOOK) % NBUF)

            @pl.when(t + LOOK < NCH)
            def _ahead():
                _gather(t + LOOK, (j + LOOK) % NBUF)

            @pl.when(t < NCH)
            def _do():
                pltpu.make_async_copy(
                    tok_hbm.at[idxg.at[pl.ds(0, GCH)]],
                    rowbuf.at[j], sem_g.at[j],
                ).wait()
                pltpu.async_copy(
                    rowbuf.at[j],
                    out_tok.at[pl.ds(row0 + t * GCH, GCH)],
                    sem_w.at[j],
                )
        return 0

    lax.fori_loop(0, (NCH + NBUF - 1) // NBUF, cbody, 0)
    for g in range(NCH - LOOK, NCH):
        _wait_write(g % NBUF)


_pruner = pl.kernel(
    _pruner_body,
    out_type=_OUT_TYPE,
    mesh=_mesh,
    scratch_types=_SCRATCH,
    compiler_params=pltpu.CompilerParams(needs_layout_passes=False),
)


def kernel(visual_tokens, scores):
    tok = visual_tokens.reshape(B * N, D)
    scr = lax.bitcast_convert_type(scores.reshape(B * N), jnp.int32)
    out_tok, out_idx, out_scr = _pruner(tok, scr)
    return (
        out_tok.reshape(B, K, D),
        out_idx.reshape(B, K),
        lax.bitcast_convert_type(out_scr, jnp.float32).reshape(B, K),
    )


# speedup vs baseline: 1.4586x; 1.0608x over previous
import functools

import jax
import jax.numpy as jnp
from jax import lax
from jax.experimental import pallas as pl
from jax.experimental.pallas import tpu as pltpu
from jax.experimental.pallas import tpu_sc as plsc

B, N, D = 8, 8192, 1024
K = N // 4
NC, NS, L = 2, 16, 16
BPC = B // NC
TPB = NS // BPC
RPT = K // TPB
GCH = 16
NBUF = 4
NV = N // L
MSB = -2147483648

_mesh = plsc.VectorSubcoreMesh(
    core_axis_name="c", subcore_axis_name="s", num_cores=NC, num_subcores=NS
)


def _keys16(sref, i):
    v = sref[pl.ds(i * L, L)]
    return v ^ (lax.shift_right_arithmetic(v, 31) & jnp.int32(0x7FFFFFFF))


_OUT_TYPE = (
    jax.ShapeDtypeStruct((B * K, D), jnp.float32),
    jax.ShapeDtypeStruct((B * K,), jnp.int32),
    jax.ShapeDtypeStruct((B * K,), jnp.int32),
)
_SCRATCH = [
    pltpu.VMEM((N,), jnp.int32),
    pltpu.VMEM((4 * N + 64,), jnp.int32),
    pltpu.VMEM((K + L,), jnp.int32),
    pltpu.VMEM((K + L,), jnp.int32),
    pltpu.VMEM((RPT,), jnp.int32),
    pltpu.VMEM((NBUF, GCH, D), jnp.float32),
    pltpu.SemaphoreType.DMA((NBUF,)),
    pltpu.SemaphoreType.DMA((NBUF,)),
]


def _pop(mask):
    return plsc.all_reduce_population_count(mask)[0]


def _pruner_body(tok_hbm, scr_hbm, out_tok, out_idx, out_scr,
                 svmem, work, oidx, oscr, idxg, rowbuf, sem_g, sem_w):
    c = lax.axis_index("c")
    s = lax.axis_index("s")

    @pl.when(s < BPC)
    def _select():
        b = c * BPC + s
        pltpu.sync_copy(scr_hbm.at[pl.ds(b * N, N)], svmem)
        lane = lax.iota(jnp.int32, L)

        def part_step0(i, carry):
            oz, oo = carry
            for u in range(2):
                v = _keys16(svmem, 2 * i + u) ^ MSB
                m1 = lax.shift_right_logical(v, 31) == 1
                plsc.store_compressed(work.at[pl.ds(oz, L)], v, mask=~m1)
                oz = oz + _pop(~m1)
                plsc.store_compressed(work.at[pl.ds(N + oo, L)], v, mask=m1)
                oo = oo + _pop(m1)
            return oz, oo

        oz0, oo0 = lax.fori_loop(
            0, NV // 2, part_step0, (jnp.int32(0), jnp.int32(0))
        )
        want0 = jnp.where(oo0 >= K, 1, 0)
        state0 = (
            jnp.where(want0 == 1, 0, oo0),
            jnp.where(want0 == 1, oo0, oz0),
            want0 << 31,
            want0 * N,
        )

        def step(j, carry):
            base, cnt, prefix, in_off = carry
            sh = 31 - j
            wb = jnp.where(in_off < 2 * N, 2 * N, 0)

            def pbody(i, carry2):
                oz, oo = carry2
                for u in range(2):
                    pos = i * 2 * L + u * L
                    v = work[pl.ds(in_off + pos, L)]
                    valid = (pos + lane) < cnt
                    m1 = valid & ((lax.shift_right_logical(v, sh) & 1) == 1)
                    m0 = valid & ~m1
                    plsc.store_compressed(work.at[pl.ds(wb + oz, L)], v, mask=m0)
                    oz = oz + _pop(m0)
                    plsc.store_compressed(
                        work.at[pl.ds(wb + N + oo, L)], v, mask=m1
                    )
                    oo = oo + _pop(m1)
                return oz, oo

            nv2 = (cnt + (2 * L - 1)) // (2 * L)
            oz, oo = lax.fori_loop(0, nv2, pbody, (jnp.int32(0), jnp.int32(0)))
            want = jnp.where(base + oo >= K, 1, 0)
            base = jnp.where(want == 1, base, base + oo)
            cnt = jnp.where(want == 1, oo, oz)
            return base, cnt, prefix | (want << sh), wb + want * N

        base, _, t_ub, _ = lax.fori_loop(1, 32, step, state0)
        t_key = t_ub ^ MSB
        need_eq = K - base

        def bbody(i, carry):
            o, e = carry
            for u in range(2):
                iv = 2 * i + u
                key = _keys16(svmem, iv)
                gt = key > t_key
                eq = key == t_key
                pe = _pop(eq)

                def slow(_):
                    eqi = jnp.where(eq, 1, 0)
                    excl = plsc.cumsum(eqi) - eqi
                    return eq & (e + excl < need_eq)

                take = lax.cond(
                    (e < need_eq) & (need_eq < e + pe),
                    slow,
                    lambda _: eq & (e < need_eq),
                    0,
                )
                keep = gt | take
                plsc.store_compressed(
                    oidx.at[pl.ds(o, L)], iv * L + lane, mask=keep
                )
                plsc.store_compressed(
                    oscr.at[pl.ds(o, L)], svmem[pl.ds(iv * L, L)], mask=keep
                )
                o = o + _pop(keep)
                e = e + pe
            return o, e

        lax.fori_loop(0, NV // 2, bbody, (jnp.int32(0), jnp.int32(0)))
        pltpu.sync_copy(oidx.at[pl.ds(0, K)], out_idx.at[pl.ds(b * K, K)])
        pltpu.sync_copy(oscr.at[pl.ds(0, K)], out_scr.at[pl.ds(b * K, K)])

    plsc.subcore_barrier()

    b2 = c * BPC + s // TPB
    row0 = b2 * K + (s % TPB) * RPT
    pltpu.sync_copy(out_idx.at[pl.ds(row0, RPT)], idxg)

    def gbody(i, _):
        idxg[pl.ds(i * L, L)] = idxg[pl.ds(i * L, L)] + b2 * N
        return 0
    lax.fori_loop(0, RPT // L, gbody, 0)

    NCH = RPT // GCH
    LOOK = 2

    def _gather(g, par):
        pltpu.async_copy(
            tok_hbm.at[idxg.at[pl.ds(g * GCH, GCH)]],
            rowbuf.at[par], sem_g.at[par],
        )

    def _wait_write(par):
        pltpu.make_async_copy(
            rowbuf.at[par], out_tok.at[pl.ds(row0, GCH)], sem_w.at[par]
        ).wait()

    for p in range(LOOK):
        _gather(p, p)

    def cbody(tt, _):
        for j in range(NBUF):
            t = tt * NBUF + j

            @pl.when((t >= LOOK) & (t < NCH))
            def _reuse():
                _wait_write((j + LOOK) % NBUF)

            @pl.when(t + LOOK < NCH)
            def _ahead():
                _gather(t + LOOK, (j + LOOK) % NBUF)

            @pl.when(t < NCH)
            def _do():
                pltpu.make_async_copy(
                    tok_hbm.at[idxg.at[pl.ds(0, GCH)]],
                    rowbuf.at[j], sem_g.at[j],
                ).wait()
                pltpu.async_copy(
                    rowbuf.at[j],
                    out_tok.at[pl.ds(row0 + t * GCH, GCH)],
                    sem_w.at[j],
                )
        return 0

    lax.fori_loop(0, (NCH + NBUF - 1) // NBUF, cbody, 0)
    for g in range(NCH - LOOK, NCH):
        _wait_write(g % NBUF)


_pruner = pl.kernel(
    _pruner_body,
    out_type=_OUT_TYPE,
    mesh=_mesh,
    scratch_types=_SCRATCH,
    compiler_params=pltpu.CompilerParams(needs_layout_passes=False),
)


def kernel(visual_tokens, scores):
    tok = visual_tokens.reshape(B * N, D)
    scr = lax.bitcast_convert_type(scores.reshape(B * N), jnp.int32)
    out_tok, out_idx, out_scr = _pruner(tok, scr)
    return (
        out_tok.reshape(B, K, D),
        out_idx.reshape(B, K),
        lax.bitcast_convert_type(out_scr, jnp.float32).reshape(B, K),
    )


# speedup vs baseline: 1.4691x; 1.0072x over previous
import functools

import jax
import jax.numpy as jnp
from jax import lax
from jax.experimental import pallas as pl
from jax.experimental.pallas import tpu as pltpu
from jax.experimental.pallas import tpu_sc as plsc

B, N, D = 8, 8192, 1024
K = N // 4
NC, NS, L = 2, 16, 16
BPC = B // NC
TPB = NS // BPC
RPT = K // TPB
GCH = 16
NBUF = 6
NV = N // L
MSB = -2147483648

_mesh = plsc.VectorSubcoreMesh(
    core_axis_name="c", subcore_axis_name="s", num_cores=NC, num_subcores=NS
)


def _keys16(sref, i):
    v = sref[pl.ds(i * L, L)]
    return v ^ (lax.shift_right_arithmetic(v, 31) & jnp.int32(0x7FFFFFFF))


_OUT_TYPE = (
    jax.ShapeDtypeStruct((B * K, D), jnp.float32),
    jax.ShapeDtypeStruct((B * K,), jnp.int32),
    jax.ShapeDtypeStruct((B * K,), jnp.int32),
)
_SCRATCH = [
    pltpu.VMEM((N,), jnp.int32),
    pltpu.VMEM((2 * N + 64,), jnp.int32),
    pltpu.VMEM((K + L,), jnp.int32),
    pltpu.VMEM((K + L,), jnp.int32),
    pltpu.VMEM((RPT,), jnp.int32),
    pltpu.VMEM((NBUF, GCH, D), jnp.float32),
    pltpu.SemaphoreType.DMA((NBUF,)),
    pltpu.SemaphoreType.DMA((NBUF,)),
]


def _pop(mask):
    return plsc.all_reduce_population_count(mask)[0]


def _pruner_body(tok_hbm, scr_hbm, out_tok, out_idx, out_scr,
                 svmem, work, oidx, oscr, idxg, rowbuf, sem_g, sem_w):
    c = lax.axis_index("c")
    s = lax.axis_index("s")

    @pl.when(s < BPC)
    def _select():
        b = c * BPC + s
        pltpu.sync_copy(scr_hbm.at[pl.ds(b * N, N)], svmem)
        lane = lax.iota(jnp.int32, L)

        def part_step0(i, carry):
            oz, oo = carry
            for u in range(2):
                v = _keys16(svmem, 2 * i + u) ^ MSB
                m1 = lax.shift_right_logical(v, 31) == 1
                plsc.store_compressed(work.at[pl.ds(oz, L)], v, mask=~m1)
                oz = oz + _pop(~m1)
                plsc.store_compressed(work.at[pl.ds(N + oo, L)], v, mask=m1)
                oo = oo + _pop(m1)
            return oz, oo

        oz0, oo0 = lax.fori_loop(
            0, NV // 2, part_step0, (jnp.int32(0), jnp.int32(0))
        )
        want0 = jnp.where(oo0 >= K, 1, 0)
        state0 = (
            jnp.where(want0 == 1, 0, oo0),
            jnp.where(want0 == 1, oo0, oz0),
            want0 << 31,
            want0 * N,
        )

        def step(j, carry):
            base, cnt, prefix, in_off = carry
            sh = 31 - j
            other = N - in_off

            def pbody(i, carry2):
                oz, oo = carry2
                for u in range(2):
                    pos = i * 2 * L + u * L
                    v = work[pl.ds(in_off + pos, L)]
                    valid = (pos + lane) < cnt
                    m1 = valid & ((lax.shift_right_logical(v, sh) & 1) == 1)
                    m0 = valid & ~m1
                    plsc.store_compressed(
                        work.at[pl.ds(in_off + oz, L)], v, mask=m0
                    )
                    oz = oz + _pop(m0)
                    plsc.store_compressed(
                        work.at[pl.ds(other + oo, L)], v, mask=m1
                    )
                    oo = oo + _pop(m1)
                return oz, oo

            nv2 = (cnt + (2 * L - 1)) // (2 * L)
            oz, oo = lax.fori_loop(0, nv2, pbody, (jnp.int32(0), jnp.int32(0)))
            want = jnp.where(base + oo >= K, 1, 0)
            base = jnp.where(want == 1, base, base + oo)
            cnt = jnp.where(want == 1, oo, oz)
            return base, cnt, prefix | (want << sh), jnp.where(want == 1, other, in_off)

        base, _, t_ub, _ = lax.fori_loop(1, 32, step, state0)
        t_key = t_ub ^ MSB
        need_eq = K - base

        def bbody(i, carry):
            o, e = carry
            for u in range(2):
                iv = 2 * i + u
                key = _keys16(svmem, iv)
                gt = key > t_key
                eq = key == t_key
                pe = _pop(eq)

                def slow(_):
                    eqi = jnp.where(eq, 1, 0)
                    excl = plsc.cumsum(eqi) - eqi
                    return eq & (e + excl < need_eq)

                take = lax.cond(
                    (e < need_eq) & (need_eq < e + pe),
                    slow,
                    lambda _: eq & (e < need_eq),
                    0,
                )
                keep = gt | take
                plsc.store_compressed(
                    oidx.at[pl.ds(o, L)], iv * L + lane, mask=keep
                )
                plsc.store_compressed(
                    oscr.at[pl.ds(o, L)], svmem[pl.ds(iv * L, L)], mask=keep
                )
                o = o + _pop(keep)
                e = e + pe
            return o, e

        lax.fori_loop(0, NV // 2, bbody, (jnp.int32(0), jnp.int32(0)))
        pltpu.sync_copy(oidx.at[pl.ds(0, K)], out_idx.at[pl.ds(b * K, K)])
        pltpu.sync_copy(oscr.at[pl.ds(0, K)], out_scr.at[pl.ds(b * K, K)])

    plsc.subcore_barrier()

    b2 = c * BPC + s // TPB
    row0 = b2 * K + (s % TPB) * RPT
    pltpu.sync_copy(out_idx.at[pl.ds(row0, RPT)], idxg)

    def gbody(i, _):
        idxg[pl.ds(i * L, L)] = idxg[pl.ds(i * L, L)] + b2 * N
        return 0
    lax.fori_loop(0, RPT // L, gbody, 0)

    NCH = RPT // GCH
    LOOK = 3

    def _gather(g, par):
        pltpu.async_copy(
            tok_hbm.at[idxg.at[pl.ds(g * GCH, GCH)]],
            rowbuf.at[par], sem_g.at[par],
        )

    def _wait_write(par):
        pltpu.make_async_copy(
            rowbuf.at[par], out_tok.at[pl.ds(row0, GCH)], sem_w.at[par]
        ).wait()

    for p in range(LOOK):
        _gather(p, p)

    def cbody(tt, _):
        for j in range(NBUF):
            t = tt * NBUF + j

            @pl.when((t >= LOOK) & (t < NCH))
            def _reuse():
                _wait_write((j + LOOK) % NBUF)

            @pl.when(t + LOOK < NCH)
            def _ahead():
                _gather(t + LOOK, (j + LOOK) % NBUF)

            @pl.when(t < NCH)
            def _do():
                pltpu.make_async_copy(
                    tok_hbm.at[idxg.at[pl.ds(0, GCH)]],
                    rowbuf.at[j], sem_g.at[j],
                ).wait()
                pltpu.async_copy(
                    rowbuf.at[j],
                    out_tok.at[pl.ds(row0 + t * GCH, GCH)],
                    sem_w.at[j],
                )
        return 0

    lax.fori_loop(0, (NCH + NBUF - 1) // NBUF, cbody, 0)
    for g in range(NCH - LOOK, NCH):
        _wait_write(g % NBUF)


_pruner = pl.kernel(
    _pruner_body,
    out_type=_OUT_TYPE,
    mesh=_mesh,
    scratch_types=_SCRATCH,
    compiler_params=pltpu.CompilerParams(needs_layout_passes=False),
)


def kernel(visual_tokens, scores):
    tok = visual_tokens.reshape(B * N, D)
    scr = lax.bitcast_convert_type(scores.reshape(B * N), jnp.int32)
    out_tok, out_idx, out_scr = _pruner(tok, scr)
    return (
        out_tok.reshape(B, K, D),
        out_idx.reshape(B, K),
        lax.bitcast_convert_type(out_scr, jnp.float32).reshape(B, K),
    )


# speedup vs baseline: 1.6267x; 1.1073x over previous
import functools

import jax
import jax.numpy as jnp
from jax import lax
from jax.experimental import pallas as pl
from jax.experimental.pallas import tpu as pltpu
from jax.experimental.pallas import tpu_sc as plsc

B, N, D = 8, 8192, 1024
K = N // 4
NC, NS, L = 2, 16, 16
BPC = B // NC
TPB = NS // BPC
RPT = K // TPB
GCH = 16
NBUF = 6
NV = N // L
NH = 2048
HSH = 21
MSB = -2147483648

_mesh = plsc.VectorSubcoreMesh(
    core_axis_name="c", subcore_axis_name="s", num_cores=NC, num_subcores=NS
)


def _keys16(sref, i):
    v = sref[pl.ds(i * L, L)]
    return v ^ (lax.shift_right_arithmetic(v, 31) & jnp.int32(0x7FFFFFFF))


_OUT_TYPE = (
    jax.ShapeDtypeStruct((B * K, D), jnp.float32),
    jax.ShapeDtypeStruct((B * K,), jnp.int32),
    jax.ShapeDtypeStruct((B * K,), jnp.int32),
)
_SCRATCH = [
    pltpu.VMEM((N,), jnp.int32),
    pltpu.VMEM((2 * N + 64,), jnp.int32),
    pltpu.VMEM((NH,), jnp.int32),
    pltpu.VMEM((K + L,), jnp.int32),
    pltpu.VMEM((K + L,), jnp.int32),
    pltpu.VMEM((RPT,), jnp.int32),
    pltpu.VMEM((NBUF, GCH, D), jnp.float32),
    pltpu.SemaphoreType.DMA((NBUF,)),
    pltpu.SemaphoreType.DMA((NBUF,)),
]


def _pop(mask):
    return plsc.all_reduce_population_count(mask)[0]


def _pruner_body(tok_hbm, scr_hbm, out_tok, out_idx, out_scr,
                 svmem, work, hist, oidx, oscr, idxg, rowbuf, sem_g, sem_w):
    c = lax.axis_index("c")
    s = lax.axis_index("s")

    @pl.when(s < BPC)
    def _select():
        b = c * BPC + s
        pltpu.sync_copy(scr_hbm.at[pl.ds(b * N, N)], svmem)
        lane = lax.iota(jnp.int32, L)

        zeros16 = jnp.zeros((L,), jnp.int32)
        ones16 = jnp.ones((L,), jnp.int32)

        def zb(i, _):
            hist[pl.ds(i * L, L)] = zeros16
            return 0
        lax.fori_loop(0, NH // L, zb, 0)

        def hb(i, _):
            for u in range(2):
                ub = _keys16(svmem, 2 * i + u) ^ MSB
                bkt = lax.shift_right_logical(ub, HSH)
                plsc.addupdate_scatter(hist, [bkt], ones16)
            return 0
        lax.fori_loop(0, NV // 2, hb, 0)

        def sb(j, carry):
            csum, bvec, svec = carry
            ii = NH // L - 1 - j
            v = hist[pl.ds(ii * L, L)]
            sfx = lax.rev(plsc.cumsum(lax.rev(v, (0,))), (0,)) + csum
            cond = (sfx >= K) & ((sfx - v) < K)
            bvec = jnp.where(cond, ii * L + lane, bvec)
            svec = jnp.where(cond, sfx - v, svec)
            return sfx[0], bvec, svec

        _, bvec, svec = lax.fori_loop(
            0, NH // L, sb,
            (jnp.int32(0), jnp.full((L,), -1, jnp.int32),
             jnp.full((L,), -1, jnp.int32)),
        )
        bstar = jnp.max(bvec)
        base0 = jnp.max(svec)

        def cb(i, o):
            for u in range(2):
                ub = _keys16(svmem, 2 * i + u) ^ MSB
                keep = lax.shift_right_logical(ub, HSH) == bstar
                plsc.store_compressed(work.at[pl.ds(o, L)], ub, mask=keep)
                o = o + _pop(keep)
            return o

        cntc = lax.fori_loop(0, NV // 2, cb, jnp.int32(0))
        state0 = (base0, cntc, bstar << HSH, jnp.int32(0))

        def step(j, carry):
            base, cnt, prefix, in_off = carry
            sh = 31 - j
            other = N - in_off

            def pbody(i, carry2):
                oz, oo = carry2
                for u in range(2):
                    pos = i * 2 * L + u * L
                    v = work[pl.ds(in_off + pos, L)]
                    valid = (pos + lane) < cnt
                    m1 = valid & ((lax.shift_right_logical(v, sh) & 1) == 1)
                    m0 = valid & ~m1
                    plsc.store_compressed(
                        work.at[pl.ds(in_off + oz, L)], v, mask=m0
                    )
                    oz = oz + _pop(m0)
                    plsc.store_compressed(
                        work.at[pl.ds(other + oo, L)], v, mask=m1
                    )
                    oo = oo + _pop(m1)
                return oz, oo

            nv2 = (cnt + (2 * L - 1)) // (2 * L)
            oz, oo = lax.fori_loop(0, nv2, pbody, (jnp.int32(0), jnp.int32(0)))
            want = jnp.where(base + oo >= K, 1, 0)
            base = jnp.where(want == 1, base, base + oo)
            cnt = jnp.where(want == 1, oo, oz)
            return base, cnt, prefix | (want << sh), jnp.where(want == 1, other, in_off)

        base, _, t_ub, _ = lax.fori_loop(32 - HSH, 32, step, state0)
        t_key = t_ub ^ MSB
        need_eq = K - base

        def bbody(i, carry):
            o, e = carry
            for u in range(2):
                iv = 2 * i + u
                key = _keys16(svmem, iv)
                gt = key > t_key
                eq = key == t_key
                pe = _pop(eq)

                def slow(_):
                    eqi = jnp.where(eq, 1, 0)
                    excl = plsc.cumsum(eqi) - eqi
                    return eq & (e + excl < need_eq)

                take = lax.cond(
                    (e < need_eq) & (need_eq < e + pe),
                    slow,
                    lambda _: eq & (e < need_eq),
                    0,
                )
                keep = gt | take
                plsc.store_compressed(
                    oidx.at[pl.ds(o, L)], iv * L + lane, mask=keep
                )
                plsc.store_compressed(
                    oscr.at[pl.ds(o, L)], svmem[pl.ds(iv * L, L)], mask=keep
                )
                o = o + _pop(keep)
                e = e + pe
            return o, e

        lax.fori_loop(0, NV // 2, bbody, (jnp.int32(0), jnp.int32(0)))
        pltpu.sync_copy(oidx.at[pl.ds(0, K)], out_idx.at[pl.ds(b * K, K)])
        pltpu.sync_copy(oscr.at[pl.ds(0, K)], out_scr.at[pl.ds(b * K, K)])

    plsc.subcore_barrier()

    b2 = c * BPC + s // TPB
    row0 = b2 * K + (s % TPB) * RPT
    pltpu.sync_copy(out_idx.at[pl.ds(row0, RPT)], idxg)

    def gbody(i, _):
        idxg[pl.ds(i * L, L)] = idxg[pl.ds(i * L, L)] + b2 * N
        return 0
    lax.fori_loop(0, RPT // L, gbody, 0)

    NCH = RPT // GCH
    LOOK = 3

    def _gather(g, par):
        pltpu.async_copy(
            tok_hbm.at[idxg.at[pl.ds(g * GCH, GCH)]],
            rowbuf.at[par], sem_g.at[par],
        )

    def _wait_write(par):
        pltpu.make_async_copy(
            rowbuf.at[par], out_tok.at[pl.ds(row0, GCH)], sem_w.at[par]
        ).wait()

    for p in range(LOOK):
        _gather(p, p)

    def cbody(tt, _):
        for j in range(NBUF):
            t = tt * NBUF + j

            @pl.when((t >= LOOK) & (t < NCH))
            def _reuse():
                _wait_write((j + LOOK) % NBUF)

            @pl.when(t + LOOK < NCH)
            def _ahead():
                _gather(t + LOOK, (j + LOOK) % NBUF)

            @pl.when(t < NCH)
            def _do():
                pltpu.make_async_copy(
                    tok_hbm.at[idxg.at[pl.ds(0, GCH)]],
                    rowbuf.at[j], sem_g.at[j],
                ).wait()
                pltpu.async_copy(
                    rowbuf.at[j],
                    out_tok.at[pl.ds(row0 + t * GCH, GCH)],
                    sem_w.at[j],
                )
        return 0

    lax.fori_loop(0, (NCH + NBUF - 1) // NBUF, cbody, 0)
    for g in range(NCH - LOOK, NCH):
        _wait_write(g % NBUF)


_pruner = pl.kernel(
    _pruner_body,
    out_type=_OUT_TYPE,
    mesh=_mesh,
    scratch_types=_SCRATCH,
    compiler_params=pltpu.CompilerParams(needs_layout_passes=False),
)


def kernel(visual_tokens, scores):
    tok = visual_tokens.reshape(B * N, D)
    scr = lax.bitcast_convert_type(scores.reshape(B * N), jnp.int32)
    out_tok, out_idx, out_scr = _pruner(tok, scr)
    return (
        out_tok.reshape(B, K, D),
        out_idx.reshape(B, K),
        lax.bitcast_convert_type(out_scr, jnp.float32).reshape(B, K),
    )
